# baseline (speedup 1.0000x reference)
import functools

import jax
import jax.numpy as jnp
from jax import lax
from jax.experimental import pallas as pl
from jax.experimental.pallas import tpu as pltpu
from jax.experimental.pallas import tpu_sc as plsc

_POSITIONS = 100000
_BATCH = 16384
_HIST = 200
_NW = 32
_COLS_PER_W = _BATCH // _NW
_CR = 8
_NCHUNKS = _HIST // _CR
_GROUPS = _CR * _COLS_PER_W // 16


def _gather_body(pos_hbm, table_hbm, out_hbm, table_v, idx_v, out_v):
    wid = lax.axis_index("s") * 2 + lax.axis_index("c")
    col0 = wid * _COLS_PER_W

    pltpu.sync_copy(table_hbm, table_v)

    def chunk_body(c, carry):
        row0 = c * _CR
        pltpu.sync_copy(
            pos_hbm.at[pl.ds(row0, _CR), pl.ds(col0, _COLS_PER_W)], idx_v)

        @plsc.parallel_loop(0, _COLS_PER_W, 16, unroll=2)
        def gather16(col):
            for r in range(_CR):
                idx = idx_v[r, pl.ds(col, 16)]
                out_v[r, pl.ds(col, 16)] = plsc.load_gather(table_v, [idx])

        pltpu.sync_copy(
            out_v, out_hbm.at[pl.ds(row0, _CR), pl.ds(col0, _COLS_PER_W)])
        return carry

    lax.fori_loop(0, _NCHUNKS, chunk_body, 0)


_gather_kernel = functools.partial(
    pl.kernel,
    out_type=jax.ShapeDtypeStruct((_HIST, _BATCH), jnp.float32),
    mesh=plsc.VectorSubcoreMesh(core_axis_name="c", subcore_axis_name="s"),
    scratch_types=[
        pltpu.VMEM((_POSITIONS,), jnp.float32),
        pltpu.VMEM((_CR, _COLS_PER_W), jnp.int32),
        pltpu.VMEM((_CR, _COLS_PER_W), jnp.float32),
    ],
    compiler_params=pltpu.CompilerParams(needs_layout_passes=False),
)(_gather_body)


@jax.jit
def kernel(positions, table):
    out_t = _gather_kernel(positions.astype(jnp.int32).T, table.reshape(-1))
    return out_t.T


# speedup vs baseline: 1.3804x; 1.3804x over previous
import functools

import jax
import jax.numpy as jnp
from jax import lax
from jax.experimental import pallas as pl
from jax.experimental.pallas import tpu as pltpu
from jax.experimental.pallas import tpu_sc as plsc

_POSITIONS = 100000
_BATCH = 16384
_HIST = 200
_NW = 32
_COLS_PER_W = _BATCH // _NW
_CR = 8
_NCHUNKS = _HIST // _CR
_GROUPS = _CR * _COLS_PER_W // 16


def _gather_body(pos_hbm, table_hbm, out_hbm, table_v,
                 idx_a, idx_b, out_a, out_b,
                 sem_ia, sem_ib, sem_oa, sem_ob):
    wid = lax.axis_index("s") * 2 + lax.axis_index("c")
    col0 = wid * _COLS_PER_W

    def pos_at(c):
        return pos_hbm.at[pl.ds(c * _CR, _CR), pl.ds(col0, _COLS_PER_W)]

    def out_at(c):
        return out_hbm.at[pl.ds(c * _CR, _CR), pl.ds(col0, _COLS_PER_W)]

    def gather(idx_v, out_v):
        @plsc.parallel_loop(0, _COLS_PER_W, 16, unroll=2)
        def gather16(col):
            for r in range(_CR):
                idx = idx_v[r, pl.ds(col, 16)]
                out_v[r, pl.ds(col, 16)] = plsc.load_gather(table_v, [idx])

    pltpu.async_copy(pos_at(0), idx_a, sem_ia)
    pltpu.sync_copy(table_hbm, table_v)

    def pair_body(k, carry):
        c = 2 * k
        pltpu.async_copy(pos_at(c + 1), idx_b, sem_ib)
        pltpu.make_async_copy(pos_at(c), idx_a, sem_ia).wait()

        @pl.when(k > 0)
        def _():
            pltpu.make_async_copy(out_a, out_at(c - 2), sem_oa).wait()

        gather(idx_a, out_a)
        pltpu.async_copy(out_a, out_at(c), sem_oa)
        pltpu.async_copy(pos_at(c + 2), idx_a, sem_ia)

        pltpu.make_async_copy(pos_at(c + 1), idx_b, sem_ib).wait()

        @pl.when(k > 0)
        def _():
            pltpu.make_async_copy(out_b, out_at(c - 1), sem_ob).wait()

        gather(idx_b, out_b)
        pltpu.async_copy(out_b, out_at(c + 1), sem_ob)
        return carry

    lax.fori_loop(0, (_NCHUNKS - 1) // 2, pair_body, 0)

    c = _NCHUNKS - 1
    pltpu.make_async_copy(pos_at(c), idx_a, sem_ia).wait()
    pltpu.make_async_copy(out_a, out_at(c - 2), sem_oa).wait()
    gather(idx_a, out_a)
    pltpu.async_copy(out_a, out_at(c), sem_oa)
    pltpu.make_async_copy(out_b, out_at(c - 1), sem_ob).wait()
    pltpu.make_async_copy(out_a, out_at(c), sem_oa).wait()


_gather_kernel = functools.partial(
    pl.kernel,
    out_type=jax.ShapeDtypeStruct((_HIST, _BATCH), jnp.float32),
    mesh=plsc.VectorSubcoreMesh(core_axis_name="c", subcore_axis_name="s"),
    scratch_types=[
        pltpu.VMEM((_POSITIONS,), jnp.float32),
        pltpu.VMEM((_CR, _COLS_PER_W), jnp.int32),
        pltpu.VMEM((_CR, _COLS_PER_W), jnp.int32),
        pltpu.VMEM((_CR, _COLS_PER_W), jnp.float32),
        pltpu.VMEM((_CR, _COLS_PER_W), jnp.float32),
        pltpu.SemaphoreType.DMA,
        pltpu.SemaphoreType.DMA,
        pltpu.SemaphoreType.DMA,
        pltpu.SemaphoreType.DMA,
    ],
    compiler_params=pltpu.CompilerParams(needs_layout_passes=False),
)(_gather_body)


@jax.jit
def kernel(positions, table):
    out_t = _gather_kernel(positions.astype(jnp.int32).T, table.reshape(-1))
    return out_t.T


# speedup vs baseline: 1.3980x; 1.0128x over previous
import functools

import jax
import jax.numpy as jnp
from jax import lax
from jax.experimental import pallas as pl
from jax.experimental.pallas import tpu as pltpu
from jax.experimental.pallas import tpu_sc as plsc

_POSITIONS = 100000
_BATCH = 16384
_HIST = 200
_NW = 32
_COLS_PER_W = _BATCH // _NW
_CR = 8
_NCHUNKS = _HIST // _CR
_GROUPS = _CR * _COLS_PER_W // 16


def _gather_body(pos_hbm, table_hbm, out_hbm, table_v,
                 idx_v, out_v, sem_i, sem_o):
    wid = lax.axis_index("s") * 2 + lax.axis_index("c")
    col0 = wid * _COLS_PER_W

    def pos_at(c):
        return pos_hbm.at[pl.ds(c * _CR, _CR), pl.ds(col0, _COLS_PER_W)]

    def out_at(c):
        return out_hbm.at[pl.ds(c * _CR, _CR), pl.ds(col0, _COLS_PER_W)]

    pltpu.async_copy(pos_at(0), idx_v.at[0], sem_i.at[0])
    pltpu.sync_copy(table_hbm, table_v)

    def chunk_body(c, carry):
        b = c & 1

        @pl.when(c + 1 < _NCHUNKS)
        def _():
            pltpu.async_copy(pos_at(c + 1), idx_v.at[1 - b], sem_i.at[1 - b])

        pltpu.make_async_copy(pos_at(c), idx_v.at[b], sem_i.at[b]).wait()

        @pl.when(c >= 2)
        def _():
            pltpu.make_async_copy(out_v.at[b], out_at(c - 2), sem_o.at[b]).wait()

        @plsc.parallel_loop(0, _COLS_PER_W, 16, unroll=2)
        def gather16(col):
            for r in range(_CR):
                idx = idx_v[b, r, pl.ds(col, 16)]
                out_v[b, r, pl.ds(col, 16)] = plsc.load_gather(table_v, [idx])

        pltpu.async_copy(out_v.at[b], out_at(c), sem_o.at[b])
        return carry

    lax.fori_loop(0, _NCHUNKS, chunk_body, 0)

    pltpu.make_async_copy(
        out_v.at[1], out_at(_NCHUNKS - 2), sem_o.at[1]).wait()
    pltpu.make_async_copy(
        out_v.at[0], out_at(_NCHUNKS - 1), sem_o.at[0]).wait()


_gather_kernel = functools.partial(
    pl.kernel,
    out_type=jax.ShapeDtypeStruct((_HIST, _BATCH), jnp.float32),
    mesh=plsc.VectorSubcoreMesh(core_axis_name="c", subcore_axis_name="s"),
    scratch_types=[
        pltpu.VMEM((_POSITIONS,), jnp.float32),
        pltpu.VMEM((2, _CR, _COLS_PER_W), jnp.int32),
        pltpu.VMEM((2, _CR, _COLS_PER_W), jnp.float32),
        pltpu.SemaphoreType.DMA((2,)),
        pltpu.SemaphoreType.DMA((2,)),
    ],
    compiler_params=pltpu.CompilerParams(needs_layout_passes=False),
)(_gather_body)


@jax.jit
def kernel(positions, table):
    out_t = _gather_kernel(positions.astype(jnp.int32).T, table.reshape(-1))
    return out_t.T


# speedup vs baseline: 1.4087x; 1.0077x over previous
import functools

import jax
import jax.numpy as jnp
from jax import lax
from jax.experimental import pallas as pl
from jax.experimental.pallas import tpu as pltpu
from jax.experimental.pallas import tpu_sc as plsc

_POSITIONS = 100000
_BATCH = 16384
_HIST = 200
_NW = 32
_COLS_PER_W = _BATCH // _NW
_CR = 8
_NCHUNKS = _HIST // _CR
_GROUPS = _CR * _COLS_PER_W // 16


def _gather_body(pos_hbm, table_hbm, out_hbm, table_v,
                 idx_v, out_v, sem_i, sem_o):
    wid = lax.axis_index("s") * 2 + lax.axis_index("c")
    col0 = wid * _COLS_PER_W

    def pos_at(c):
        return pos_hbm.at[pl.ds(c * _CR, _CR), pl.ds(col0, _COLS_PER_W)]

    def out_at(c):
        return out_hbm.at[pl.ds(c * _CR, _CR), pl.ds(col0, _COLS_PER_W)]

    pltpu.async_copy(pos_at(0), idx_v.at[0], sem_i.at[0])
    pltpu.sync_copy(table_hbm.at[0], table_v)

    def chunk_body(c, carry):
        b = c & 1

        @pl.when(c + 1 < _NCHUNKS)
        def _():
            pltpu.async_copy(pos_at(c + 1), idx_v.at[1 - b], sem_i.at[1 - b])

        pltpu.make_async_copy(pos_at(c), idx_v.at[b], sem_i.at[b]).wait()

        @pl.when(c >= 2)
        def _():
            pltpu.make_async_copy(out_v.at[b], out_at(c - 2), sem_o.at[b]).wait()

        @plsc.parallel_loop(0, _COLS_PER_W, 16, unroll=2)
        def gather16(col):
            for r in range(_CR):
                idx = idx_v[b, r, pl.ds(col, 16)]
                out_v[b, r, pl.ds(col, 16)] = plsc.load_gather(table_v, [idx])

        pltpu.async_copy(out_v.at[b], out_at(c), sem_o.at[b])
        return carry

    lax.fori_loop(0, _NCHUNKS, chunk_body, 0)

    pltpu.make_async_copy(
        out_v.at[1], out_at(_NCHUNKS - 2), sem_o.at[1]).wait()
    pltpu.make_async_copy(
        out_v.at[0], out_at(_NCHUNKS - 1), sem_o.at[0]).wait()


_gather_kernel = functools.partial(
    pl.kernel,
    out_type=jax.ShapeDtypeStruct((_HIST, _BATCH), jnp.float32),
    mesh=plsc.VectorSubcoreMesh(core_axis_name="c", subcore_axis_name="s"),
    scratch_types=[
        pltpu.VMEM((_POSITIONS,), jnp.float32),
        pltpu.VMEM((2, _CR, _COLS_PER_W), jnp.int32),
        pltpu.VMEM((2, _CR, _COLS_PER_W), jnp.float32),
        pltpu.SemaphoreType.DMA((2,)),
        pltpu.SemaphoreType.DMA((2,)),
    ],
    compiler_params=pltpu.CompilerParams(needs_layout_passes=False),
)(_gather_body)


@jax.jit
def kernel(positions, table):
    out_t = _gather_kernel(positions.astype(jnp.int32).T, table.T)
    return out_t.T
